# Initial kernel scaffold; baseline (speedup 1.0000x reference)
import jax
import jax.numpy as jnp
from jax.experimental import pallas as pl


def kernel(g_x, g_edge_index, lg_x, lg_edge_index, lg_edge_attr, params_node, params_edge):
    raise NotImplementedError("write your pallas kernel here")


# speedup vs baseline: 2.1534x; 2.1534x over previous
import jax
import jax.numpy as jnp
from jax.experimental import pallas as pl

D = 128


def _pick_block(n, cands=(512, 400, 256, 200, 128, 80, 64, 40, 16, 8)):
    for c in cands:
        if n % c == 0:
            return c
    raise ValueError(n)


def _dense_body(x_ref, w_ref, o_ref):
    o_ref[...] = jnp.dot(x_ref[...], w_ref[...],
                         preferred_element_type=jnp.float32)


def _dense(x, w):
    n, k = x.shape[0], w.shape[1]
    br = _pick_block(n)
    return pl.pallas_call(
        _dense_body,
        grid=(n // br,),
        in_specs=[
            pl.BlockSpec((br, D), lambda i: (i, 0)),
            pl.BlockSpec((D, k), lambda i: (0, 0)),
        ],
        out_specs=pl.BlockSpec((br, k), lambda i: (i, 0)),
        out_shape=jax.ShapeDtypeStruct((n, k), jnp.float32),
    )(x, w)


def _edge_ew_body(ag_ref, bg_ref, e0_ref, ug_ref, db_ref, eb_ref,
                  sig_ref, msg_ref):
    gate = ag_ref[...] + bg_ref[...] + db_ref[...] + e0_ref[...] + eb_ref[...]
    sig = jax.nn.silu(gate)
    sig_ref[...] = sig
    msg_ref[...] = sig * ug_ref[...]


def _edge_ew(ag, bg, e0, ug, db, eb):
    ne = ag.shape[0]
    br = _pick_block(ne)
    return pl.pallas_call(
        _edge_ew_body,
        grid=(ne // br,),
        in_specs=[pl.BlockSpec((br, D), lambda i: (i, 0))] * 4
                 + [pl.BlockSpec((1, D), lambda i: (0, 0))] * 2,
        out_specs=[pl.BlockSpec((br, D), lambda i: (i, 0))] * 2,
        out_shape=[jax.ShapeDtypeStruct((ne, D), jnp.float32)] * 2,
    )(ag, bg, e0, ug, db, eb)


def _div_body(m_ref, n_ref, o_ref):
    o_ref[...] = m_ref[...] / n_ref[...]


def _div(msg, normg):
    ne = msg.shape[0]
    br = _pick_block(ne)
    return pl.pallas_call(
        _div_body,
        grid=(ne // br,),
        in_specs=[pl.BlockSpec((br, D), lambda i: (i, 0))] * 2,
        out_specs=pl.BlockSpec((br, D), lambda i: (i, 0)),
        out_shape=jax.ShapeDtypeStruct((ne, D), jnp.float32),
    )(msg, normg)


def _sum_body(s_ref, a_ref, o_ref):
    o_ref[...] = s_ref[...] + a_ref[...]


def _sum2(s, aggr):
    n = s.shape[0]
    br = _pick_block(n)
    return pl.pallas_call(
        _sum_body,
        grid=(n // br,),
        in_specs=[pl.BlockSpec((br, D), lambda i: (i, 0))] * 2,
        out_specs=pl.BlockSpec((br, D), lambda i: (i, 0)),
        out_shape=jax.ShapeDtypeStruct((n, D), jnp.float32),
    )(s, aggr)


def _bn_body(op_ref, m_ref, dn_ref, g_ref, be_ref, x_ref, o_ref):
    y = (op_ref[...] - m_ref[...]) / dn_ref[...] * g_ref[...] + be_ref[...]
    o_ref[...] = jax.nn.silu(y) + x_ref[...]


def _bn_silu_res(out_pre, mean, denom, gamma, beta, x):
    n = out_pre.shape[0]
    br = _pick_block(n)
    return pl.pallas_call(
        _bn_body,
        grid=(n // br,),
        in_specs=[
            pl.BlockSpec((br, D), lambda i: (i, 0)),
            pl.BlockSpec((1, D), lambda i: (0, 0)),
            pl.BlockSpec((1, D), lambda i: (0, 0)),
            pl.BlockSpec((1, D), lambda i: (0, 0)),
            pl.BlockSpec((1, D), lambda i: (0, 0)),
            pl.BlockSpec((br, D), lambda i: (i, 0)),
        ],
        out_specs=pl.BlockSpec((br, D), lambda i: (i, 0)),
        out_shape=jax.ShapeDtypeStruct((n, D), jnp.float32),
    )(out_pre, mean, denom, gamma, beta, x)


def _egc_layer(x, src, dst, e0, p):
    n = x.shape[0]
    wcat = jnp.concatenate(
        [p['src_gate_w'], p['dst_gate_w'], p['dst_update_w'],
         p['src_update_w']], axis=1)
    absu = _dense(x, wcat)
    a = absu[:, :D] + p['src_gate_b']
    b0 = absu[:, D:2 * D]
    u = absu[:, 2 * D:3 * D] + p['dst_update_b']
    s = absu[:, 3 * D:] + p['src_update_b']
    sig, msg = _edge_ew(a[dst], b0[src], e0, u[src],
                        p['dst_gate_b'][None, :], p['edge_gate_b'][None, :])
    norm = jax.ops.segment_sum(sig, dst, num_segments=n) + 1e-08
    msg = _div(msg, jnp.take(norm, dst, axis=0))
    aggr = jax.ops.segment_sum(msg, dst, num_segments=n)
    out_pre = _sum2(s, aggr)
    mean = jnp.mean(out_pre, axis=0)[None, :]
    var = jnp.mean((out_pre - mean) ** 2, axis=0)[None, :]
    denom = jnp.sqrt(var + 1e-05)
    return _bn_silu_res(out_pre, mean, denom, p['bn_gamma'][None, :],
                        p['bn_beta'][None, :], x)


def kernel(g_x, g_edge_index, lg_x, lg_edge_index, lg_edge_attr,
           params_node, params_edge):
    lg_src = lg_edge_index[0]
    lg_dst = lg_edge_index[1]
    g_src = g_edge_index[0]
    g_dst = g_edge_index[1]
    e01 = _dense(lg_edge_attr, params_edge['edge_gate_w'])
    ea = _egc_layer(lg_x, lg_src, lg_dst, e01, params_edge)
    e02 = _dense(ea, params_node['edge_gate_w'])
    xn = _egc_layer(g_x, g_src, g_dst, e02, params_node)
    return (xn, ea)
